# baseline (speedup 1.0000x reference)
import functools

import jax
import jax.numpy as jnp
from jax import lax
from jax.experimental import pallas as pl
from jax.experimental.pallas import tpu as pltpu
from jax.experimental.pallas import tpu_sc as plsc

_NC = 2
_NS = 16
_NW = _NC * _NS
_K = 128
_L = 16
_CP = 128


def _interp_sc(tab, meta, wts, N, C):
    per_w = N // _NW
    chunks = per_w // _K
    mesh = plsc.VectorSubcoreMesh(core_axis_name="c", subcore_axis_name="s")

    taps_t = pltpu.VMEM((_K, _CP), jnp.int32)
    nb = C // 32

    @functools.partial(
        pl.kernel,
        out_type=jax.ShapeDtypeStruct((N, C // 2), jnp.int32),
        mesh=mesh,
        compiler_params=pltpu.CompilerParams(needs_layout_passes=False),
        scratch_types=[
            pltpu.VMEM((2, _K), jnp.int32),
            pltpu.VMEM((2, _K), jnp.int32),
            pltpu.VMEM((4, _K), jnp.float32),
            pltpu.VMEM((4, _K), jnp.float32),
            taps_t, taps_t, taps_t, taps_t,
            pltpu.VMEM((_K, C // 2), jnp.int32),
            pltpu.VMEM((_K, C // 2), jnp.int32),
            pltpu.SemaphoreType.DMA,
            pltpu.SemaphoreType.DMA,
            pltpu.SemaphoreType.DMA,
            pltpu.SemaphoreType.DMA,
            pltpu.SemaphoreType.DMA,
            pltpu.SemaphoreType.DMA,
            pltpu.SemaphoreType.DMA,
            pltpu.SemaphoreType.DMA,
        ],
    )
    def k(tab_hbm, meta_hbm, w_hbm, out_hbm, m0, m1, w0, w1,
          ta0, ta1, tb0, tb1, ov0, ov1,
          sem_ga, sem_gb, sem_ma, sem_mb, sem_wa, sem_wb, sem_oa, sem_ob):
        wid = lax.axis_index("s") * _NC + lax.axis_index("c")
        base0 = wid * per_w
        metas = (m0, m1)
        wvs = (w0, w1)
        taps = ((ta0, ta1), (tb0, tb1))
        outs = (ov0, ov1)
        gsems = (sem_ga, sem_gb)
        msems = (sem_ma, sem_mb)
        wsems = (sem_wa, sem_wb)
        osems = (sem_oa, sem_ob)

        def wait_gathers(p):
            for t in range(2):
                pltpu.make_async_copy(
                    tab_hbm.at[metas[p].at[t]], taps[p][t], gsems[p]).wait()

        def issue_gathers(p):
            for t in range(2):
                pltpu.async_copy(
                    tab_hbm.at[metas[p].at[t]], taps[p][t], gsems[p])

        def issue_meta(g, p):
            pltpu.async_copy(meta_hbm.at[wid, g], metas[p], msems[p])

        def wait_meta(p):
            pltpu.make_async_copy(
                meta_hbm.at[wid, 0], metas[p], msems[p]).wait()

        def issue_w(g, p):
            pltpu.async_copy(w_hbm.at[wid, g], wvs[p], wsems[p])

        def wait_w(p):
            pltpu.make_async_copy(
                w_hbm.at[wid, 0], wvs[p], wsems[p]).wait()

        def wait_out(p):
            pltpu.make_async_copy(
                outs[p], out_hbm.at[pl.ds(0, _K)], osems[p]).wait()

        def do_chunk(g, i, p, guard_out):
            q = 1 - p
            wait_gathers(p)
            wait_meta(q)
            issue_gathers(q)
            issue_meta(g + 2, p)
            wait_w(p)
            if guard_out is None:
                wait_out(p)
            else:
                @pl.when(guard_out)
                def _():
                    wait_out(p)
            top, bot = taps[p]
            w_v = wvs[p]
            out_v = outs[p]

            def grp_body(g2, carry):
                bp = g2 * _L
                wv = [w_v[t, pl.ds(bp, _L)] for t in range(4)]
                def bsplat(s):
                    v = jnp.full((_L,), s, jnp.float32)
                    return plsc.pack(v, v, format=plsc.PackFormat.INTERLEAVED)

                for ii in range(_L):
                    pix = bp + ii
                    wtl = bsplat(wv[0][ii])
                    wtr = bsplat(wv[1][ii])
                    wbl = bsplat(wv[2][ii])
                    wbr = bsplat(wv[3][ii])
                    for j in range(nb):
                        sa = pl.ds(j * _L, _L)
                        sb = pl.ds(nb * _L + j * _L, _L)
                        vtl = plsc.bitcast(top[pix, sa], jnp.bfloat16)
                        vtr = plsc.bitcast(top[pix, sb], jnp.bfloat16)
                        vbl = plsc.bitcast(bot[pix, sa], jnp.bfloat16)
                        vbr = plsc.bitcast(bot[pix, sb], jnp.bfloat16)
                        acc = wtl * vtl + wtr * vtr
                        acc = acc + wbl * vbl + wbr * vbr
                        out_v[pix, sa] = plsc.bitcast(acc, jnp.int32)
                return carry

            lax.fori_loop(0, _K // _L, grp_body, 0)
            issue_w(g + 2, p)
            pltpu.async_copy(
                out_v, out_hbm.at[pl.ds(base0 + g * _K, _K)], osems[p])

        issue_meta(0, 0)
        issue_meta(1, 1)
        issue_w(0, 0)
        issue_w(1, 1)
        wait_meta(0)
        issue_gathers(0)

        def pair_body(i, carry):
            do_chunk(2 * i, i, 0, i > 0)
            do_chunk(2 * i + 1, i, 1, i > 0)
            return carry

        lax.fori_loop(0, chunks // 2, pair_body, 0)
        wait_gathers(0)
        wait_meta(1)
        wait_w(0)
        wait_w(1)
        wait_out(0)
        wait_out(1)

    return k(tab, meta, wts)


def kernel(input, flow, residual):
    x = input
    B, C, H, W = x.shape
    N = B * H * W
    per_w = N // _NW
    chunks = per_w // _K

    gy = jnp.linspace(-1.0 + 1.0 / H, 1.0 - 1.0 / H, H, dtype=x.dtype)
    gx = jnp.linspace(-1.0 + 1.0 / W, 1.0 - 1.0 / W, W, dtype=x.dtype)
    grid_x = gx[None, None, :] + flow[:, 0]
    grid_y = gy[None, :, None] + flow[:, 1]
    grid_x = jnp.remainder(grid_x + 1.0, 2.0) - 1.0
    real_x = (grid_x + 1.0) * (W * 0.5) - 0.5
    real_y = (grid_y + 1.0) * (H * 0.5) - 0.5
    x0f = jnp.floor(real_x)
    y0f = jnp.floor(real_y)
    dx = real_x - x0f
    dy = real_y - y0f
    ix0 = x0f.astype(jnp.int32)
    iy0 = y0f.astype(jnp.int32)
    boff = (jnp.arange(B, dtype=jnp.int32) * (H * W))[:, None, None]

    def wvalid(iy, ix, w):
        valid = (ix >= 0) & (ix < W) & (iy >= 0) & (iy < H)
        return (w * valid).reshape(N)

    w_tl = wvalid(iy0, ix0, (1.0 - dx) * (1.0 - dy))
    w_tr = wvalid(iy0, ix0 + 1, dx * (1.0 - dy))
    w_bl = wvalid(iy0 + 1, ix0, (1.0 - dx) * dy)
    w_br = wvalid(iy0 + 1, ix0 + 1, dx * dy)

    def pair_row(iy):
        p = (boff + jnp.clip(iy, 0, H - 1) * W + ix0).reshape(N)
        return jnp.where((p & 1) == 0, p >> 1, (N // 2) + ((p + 1) >> 1))

    r_top = pair_row(iy0)
    r_bot = pair_row(iy0 + 1)

    meta = jnp.stack([r_top, r_bot])
    meta = meta.reshape(2, _NW, chunks, _K).transpose(1, 2, 0, 3)
    meta = jnp.pad(meta, ((0, 0), (0, 2), (0, 0), (0, 0)))
    wts = jnp.stack([w_tl, w_tr, w_bl, w_br])
    wts = wts.reshape(4, _NW, chunks, _K).transpose(1, 2, 0, 3)
    wts = jnp.pad(wts, ((0, 0), (0, 2), (0, 0), (0, 0)))

    xt = jnp.transpose(x, (0, 2, 3, 1)).reshape(N, C).astype(jnp.bfloat16)
    t0 = lax.bitcast_convert_type(
        xt.reshape(N // 2, C, 2), jnp.int32)
    pad_px = jnp.zeros((1, C), jnp.bfloat16)
    t1 = lax.bitcast_convert_type(
        jnp.concatenate([pad_px, xt, pad_px], axis=0)
        .reshape(N // 2 + 1, C, 2), jnp.int32)
    tab = jnp.concatenate([t0, t1], axis=0)
    tab = jnp.pad(tab, ((0, 0), (0, _CP - C)))
    out_i = _interp_sc(tab, meta, wts, N, C)
    out_t = lax.bitcast_convert_type(out_i, jnp.bfloat16).reshape(N, C)
    out_t = out_t.astype(jnp.float32)
    return out_t.reshape(B, H, W, C).transpose(0, 3, 1, 2) + residual


# speedup vs baseline: 7.9775x; 7.9775x over previous
import functools

import jax
import jax.numpy as jnp
from jax import lax
from jax.experimental import pallas as pl
from jax.experimental.pallas import tpu as pltpu
from jax.experimental.pallas import tpu_sc as plsc

_NC = 2
_NS = 16
_NW = _NC * _NS
_K = 128
_L = 16
_CP = 128


def _interp_sc(tab, meta, wts, N, C):
    per_w = N // _NW
    chunks = per_w // _K
    mesh = plsc.VectorSubcoreMesh(core_axis_name="c", subcore_axis_name="s")

    taps_t = pltpu.VMEM((_K, _CP), jnp.int32)
    nb = C // 32

    @functools.partial(
        pl.kernel,
        out_type=jax.ShapeDtypeStruct((N, C // 2), jnp.int32),
        mesh=mesh,
        compiler_params=pltpu.CompilerParams(needs_layout_passes=False),
        scratch_types=[
            pltpu.VMEM((2, _K), jnp.int32),
            pltpu.VMEM((2, _K), jnp.int32),
            pltpu.VMEM((4, _K), jnp.float32),
            pltpu.VMEM((4, _K), jnp.float32),
            taps_t, taps_t, taps_t, taps_t,
            pltpu.VMEM((_K, C // 2), jnp.int32),
            pltpu.VMEM((_K, C // 2), jnp.int32),
            pltpu.SemaphoreType.DMA,
            pltpu.SemaphoreType.DMA,
            pltpu.SemaphoreType.DMA,
            pltpu.SemaphoreType.DMA,
            pltpu.SemaphoreType.DMA,
            pltpu.SemaphoreType.DMA,
            pltpu.SemaphoreType.DMA,
            pltpu.SemaphoreType.DMA,
        ],
    )
    def k(tab_hbm, meta_hbm, w_hbm, out_hbm, m0, m1, w0, w1,
          ta0, ta1, tb0, tb1, ov0, ov1,
          sem_ga, sem_gb, sem_ma, sem_mb, sem_wa, sem_wb, sem_oa, sem_ob):
        wid = lax.axis_index("s") * _NC + lax.axis_index("c")
        base0 = wid * per_w
        metas = (m0, m1)
        wvs = (w0, w1)
        taps = ((ta0, ta1), (tb0, tb1))
        outs = (ov0, ov1)
        gsems = (sem_ga, sem_gb)
        msems = (sem_ma, sem_mb)
        wsems = (sem_wa, sem_wb)
        osems = (sem_oa, sem_ob)

        def wait_gathers(p):
            for t in range(2):
                pltpu.make_async_copy(
                    tab_hbm.at[metas[p].at[t]], taps[p][t], gsems[p]).wait()

        def issue_gathers(p):
            for t in range(2):
                pltpu.async_copy(
                    tab_hbm.at[metas[p].at[t]], taps[p][t], gsems[p])

        def issue_meta(g, p):
            pltpu.async_copy(meta_hbm.at[wid, g], metas[p], msems[p])

        def wait_meta(p):
            pltpu.make_async_copy(
                meta_hbm.at[wid, 0], metas[p], msems[p]).wait()

        def issue_w(g, p):
            pltpu.async_copy(w_hbm.at[wid, g], wvs[p], wsems[p])

        def wait_w(p):
            pltpu.make_async_copy(
                w_hbm.at[wid, 0], wvs[p], wsems[p]).wait()

        def wait_out(p):
            pltpu.make_async_copy(
                outs[p], out_hbm.at[pl.ds(0, _K)], osems[p]).wait()

        def do_chunk(g, i, p, guard_out):
            q = 1 - p
            wait_gathers(p)
            wait_meta(q)
            issue_gathers(q)
            issue_meta(g + 2, p)
            wait_w(p)
            if guard_out is None:
                wait_out(p)
            else:
                @pl.when(guard_out)
                def _():
                    wait_out(p)
            top, bot = taps[p]
            w_v = wvs[p]
            out_v = outs[p]

            def grp_body(g2, carry):
                bp = g2 * _L
                wv = [w_v[t, pl.ds(bp, _L)] for t in range(4)]
                def bsplat(s):
                    v = jnp.full((_L,), s, jnp.float32)
                    return plsc.pack(v, v, format=plsc.PackFormat.INTERLEAVED)

                for ii in range(_L):
                    pix = bp + ii
                    wtl = bsplat(wv[0][ii])
                    wtr = bsplat(wv[1][ii])
                    wbl = bsplat(wv[2][ii])
                    wbr = bsplat(wv[3][ii])
                    for j in range(nb):
                        sa = pl.ds(j * _L, _L)
                        sb = pl.ds(nb * _L + j * _L, _L)
                        vtl = plsc.bitcast(top[pix, sa], jnp.bfloat16)
                        vtr = plsc.bitcast(top[pix, sb], jnp.bfloat16)
                        vbl = plsc.bitcast(bot[pix, sa], jnp.bfloat16)
                        vbr = plsc.bitcast(bot[pix, sb], jnp.bfloat16)
                        acc = wtl * vtl + wtr * vtr
                        acc = acc + wbl * vbl + wbr * vbr
                        out_v[pix, sa] = plsc.bitcast(acc, jnp.int32)
                return carry

            lax.fori_loop(0, _K // _L, grp_body, 0)
            issue_w(g + 2, p)
            pltpu.async_copy(
                out_v, out_hbm.at[pl.ds(base0 + g * _K, _K)], osems[p])

        issue_meta(0, 0)
        issue_meta(1, 1)
        issue_w(0, 0)
        issue_w(1, 1)
        wait_meta(0)
        issue_gathers(0)

        def pair_body(i, carry):
            do_chunk(2 * i, i, 0, i > 0)
            do_chunk(2 * i + 1, i, 1, i > 0)
            return carry

        lax.fori_loop(0, chunks // 2, pair_body, 0)
        wait_gathers(0)
        wait_meta(1)
        wait_w(0)
        wait_w(1)
        wait_out(0)
        wait_out(1)

    return k(tab, meta, wts)


def kernel(input, flow, residual):
    x = input
    B, C, H, W = x.shape
    N = B * H * W
    per_w = N // _NW
    chunks = per_w // _K

    gy = jnp.linspace(-1.0 + 1.0 / H, 1.0 - 1.0 / H, H, dtype=x.dtype)
    gx = jnp.linspace(-1.0 + 1.0 / W, 1.0 - 1.0 / W, W, dtype=x.dtype)
    grid_x = gx[None, None, :] + flow[:, 0]
    grid_y = gy[None, :, None] + flow[:, 1]
    grid_x = jnp.remainder(grid_x + 1.0, 2.0) - 1.0
    real_x = (grid_x + 1.0) * (W * 0.5) - 0.5
    real_y = (grid_y + 1.0) * (H * 0.5) - 0.5
    x0f = jnp.floor(real_x)
    y0f = jnp.floor(real_y)
    dx = real_x - x0f
    dy = real_y - y0f
    ix0 = x0f.astype(jnp.int32)
    iy0 = y0f.astype(jnp.int32)
    boff = (jnp.arange(B, dtype=jnp.int32) * (H * W))[:, None, None]

    def wvalid(iy, ix, w):
        valid = (ix >= 0) & (ix < W) & (iy >= 0) & (iy < H)
        return (w * valid).reshape(N)

    w_tl = wvalid(iy0, ix0, (1.0 - dx) * (1.0 - dy))
    w_tr = wvalid(iy0, ix0 + 1, dx * (1.0 - dy))
    w_bl = wvalid(iy0 + 1, ix0, (1.0 - dx) * dy)
    w_br = wvalid(iy0 + 1, ix0 + 1, dx * dy)

    def pair_row(iy):
        p = (boff + jnp.clip(iy, 0, H - 1) * W + ix0).reshape(N)
        return jnp.where((p & 1) == 0, p >> 1, (N // 2) + ((p + 1) >> 1))

    r_top = pair_row(iy0)
    r_bot = pair_row(iy0 + 1)

    meta = jnp.stack([r_top, r_bot])
    meta = meta.reshape(2, _NW, chunks, _K).transpose(1, 2, 0, 3)
    meta = jnp.pad(meta, ((0, 0), (0, 2), (0, 0), (0, 0)))
    wts = jnp.stack([w_tl, w_tr, w_bl, w_br])
    wts = wts.reshape(4, _NW, chunks, _K).transpose(1, 2, 0, 3)
    wts = jnp.pad(wts, ((0, 0), (0, 2), (0, 0), (0, 0)))

    xt = jnp.transpose(x, (0, 2, 3, 1)).reshape(N, C)
    u = lax.bitcast_convert_type(xt, jnp.uint32)
    r = (u + jnp.uint32(0x7FFF) + ((u >> 16) & jnp.uint32(1))) >> 16
    hc = C // 2
    packed = lax.bitcast_convert_type(
        r[:, :hc] | (r[:, hc:] << 16), jnp.int32)
    t0 = packed.reshape(N // 2, C)
    pad_px = jnp.zeros((1, hc), jnp.int32)
    t1 = jnp.concatenate([pad_px, packed, pad_px], axis=0).reshape(
        N // 2 + 1, C)
    tab = jnp.concatenate([t0, t1], axis=0)
    tab = jnp.pad(tab, ((0, 0), (0, _CP - C)))
    out_i = _interp_sc(tab, meta, wts, N, C)
    oi = lax.bitcast_convert_type(out_i, jnp.uint32)
    lo_f = lax.bitcast_convert_type(oi << 16, jnp.float32)
    hi_f = lax.bitcast_convert_type(oi & jnp.uint32(0xFFFF0000), jnp.float32)
    out_t = jnp.concatenate([lo_f, hi_f], axis=1)
    return out_t.reshape(B, H, W, C).transpose(0, 3, 1, 2) + residual


# speedup vs baseline: 12.5017x; 1.5671x over previous
import functools

import jax
import jax.numpy as jnp
from jax import lax
from jax.experimental import pallas as pl
from jax.experimental.pallas import tpu as pltpu
from jax.experimental.pallas import tpu_sc as plsc

_NC = 2
_NS = 16
_NW = _NC * _NS
_K = 64
_L = 16
_CP = 128


def _interp_sc(xt, meta, wts, N, C):
    per_w = N // _NW
    chunks = per_w // _K
    mesh = plsc.VectorSubcoreMesh(core_axis_name="c", subcore_axis_name="s")

    taps_t = pltpu.VMEM((_K, _CP), jnp.float32)

    @functools.partial(
        pl.kernel,
        out_type=jax.ShapeDtypeStruct((N, C), jnp.float32),
        mesh=mesh,
        compiler_params=pltpu.CompilerParams(use_tc_tiling_on_sc=True),
        scratch_types=[
            pltpu.VMEM((4, _K), jnp.int32),
            pltpu.VMEM((4, _K), jnp.int32),
            pltpu.VMEM((4, _K), jnp.float32),
            pltpu.VMEM((4, _K), jnp.float32),
            taps_t, taps_t, taps_t, taps_t,
            taps_t, taps_t, taps_t, taps_t,
            pltpu.VMEM((_K, C), jnp.float32),
            pltpu.VMEM((_K, C), jnp.float32),
            pltpu.SemaphoreType.DMA,
            pltpu.SemaphoreType.DMA,
            pltpu.SemaphoreType.DMA,
            pltpu.SemaphoreType.DMA,
            pltpu.SemaphoreType.DMA,
            pltpu.SemaphoreType.DMA,
            pltpu.SemaphoreType.DMA,
            pltpu.SemaphoreType.DMA,
        ],
    )
    def k(xt_hbm, meta_hbm, w_hbm, out_hbm, m0, m1, w0, w1,
          ta0, ta1, ta2, ta3, tb0, tb1, tb2, tb3, ov0, ov1,
          sem_ga, sem_gb, sem_ma, sem_mb, sem_wa, sem_wb, sem_oa, sem_ob):
        wid = lax.axis_index("s") * _NC + lax.axis_index("c")
        base0 = wid * per_w
        metas = (m0, m1)
        wvs = (w0, w1)
        taps = ((ta0, ta1, ta2, ta3), (tb0, tb1, tb2, tb3))
        outs = (ov0, ov1)
        gsems = (sem_ga, sem_gb)
        msems = (sem_ma, sem_mb)
        wsems = (sem_wa, sem_wb)
        osems = (sem_oa, sem_ob)

        def wait_gathers(p):
            for t in range(4):
                pltpu.make_async_copy(
                    xt_hbm.at[metas[p].at[t]], taps[p][t], gsems[p]).wait()

        def issue_gathers(p):
            for t in range(4):
                pltpu.async_copy(
                    xt_hbm.at[metas[p].at[t]], taps[p][t], gsems[p])

        def issue_meta(g, p):
            pltpu.async_copy(meta_hbm.at[wid, g], metas[p], msems[p])

        def wait_meta(p):
            pltpu.make_async_copy(
                meta_hbm.at[wid, 0], metas[p], msems[p]).wait()

        def issue_w(g, p):
            pltpu.async_copy(w_hbm.at[wid, g], wvs[p], wsems[p])

        def wait_w(p):
            pltpu.make_async_copy(
                w_hbm.at[wid, 0], wvs[p], wsems[p]).wait()

        def wait_out(p):
            pltpu.make_async_copy(
                outs[p], out_hbm.at[pl.ds(0, _K)], osems[p]).wait()

        def do_chunk(g, i, p, guard_out):
            q = 1 - p
            wait_gathers(p)
            wait_meta(q)
            issue_gathers(q)
            issue_meta(g + 2, p)
            wait_w(p)
            if guard_out is None:
                wait_out(p)
            else:
                @pl.when(guard_out)
                def _():
                    wait_out(p)
            tp = taps[p]
            w_v = wvs[p]
            out_v = outs[p]

            def grp_body(g2, carry):
                bp = g2 * _L
                wv = [w_v[t, pl.ds(bp, _L)] for t in range(4)]
                for ii in range(_L):
                    pix = bp + ii
                    ws = [wv[t][ii] for t in range(4)]
                    for j in range(C // _L):
                        s = pl.ds(j * _L, _L)
                        acc = ws[0] * tp[0][pix, s] + ws[1] * tp[1][pix, s]
                        acc = acc + ws[2] * tp[2][pix, s] + ws[3] * tp[3][pix, s]
                        out_v[pix, s] = acc
                return carry

            lax.fori_loop(0, _K // _L, grp_body, 0)
            issue_w(g + 2, p)
            pltpu.async_copy(
                out_v, out_hbm.at[pl.ds(base0 + g * _K, _K)], osems[p])

        issue_meta(0, 0)
        issue_meta(1, 1)
        issue_w(0, 0)
        issue_w(1, 1)
        wait_meta(0)
        issue_gathers(0)

        def pair_body(i, carry):
            do_chunk(2 * i, i, 0, i > 0)
            do_chunk(2 * i + 1, i, 1, i > 0)
            return carry

        lax.fori_loop(0, chunks // 2, pair_body, 0)
        wait_gathers(0)
        wait_meta(1)
        wait_w(0)
        wait_w(1)
        wait_out(0)
        wait_out(1)

    return k(xt, meta, wts)


def kernel(input, flow, residual):
    x = input
    B, C, H, W = x.shape
    N = H * W
    per_w = N // _NW
    chunks = per_w // _K

    gy = jnp.linspace(-1.0 + 1.0 / H, 1.0 - 1.0 / H, H, dtype=x.dtype)
    gx = jnp.linspace(-1.0 + 1.0 / W, 1.0 - 1.0 / W, W, dtype=x.dtype)
    grid_x = gx[None, None, :] + flow[:, 0]
    grid_y = gy[None, :, None] + flow[:, 1]
    grid_x = jnp.remainder(grid_x + 1.0, 2.0) - 1.0
    real_x = (grid_x + 1.0) * (W * 0.5) - 0.5
    real_y = (grid_y + 1.0) * (H * 0.5) - 0.5
    x0f = jnp.floor(real_x)
    y0f = jnp.floor(real_y)
    dx = real_x - x0f
    dy = real_y - y0f
    ix0 = x0f.astype(jnp.int32)
    iy0 = y0f.astype(jnp.int32)

    def idx_w(iy, ix, w):
        valid = (ix >= 0) & (ix < W) & (iy >= 0) & (iy < H)
        iyc = jnp.clip(iy, 0, H - 1)
        ixc = jnp.clip(ix, 0, W - 1)
        idx = iyc * W + ixc
        return idx.reshape(B, N), (w * valid).reshape(B, N)

    i_tl, w_tl = idx_w(iy0, ix0, (1.0 - dx) * (1.0 - dy))
    i_tr, w_tr = idx_w(iy0, ix0 + 1, dx * (1.0 - dy))
    i_bl, w_bl = idx_w(iy0 + 1, ix0, (1.0 - dx) * dy)
    i_br, w_br = idx_w(iy0 + 1, ix0 + 1, dx * dy)
    idx4 = jnp.stack([i_tl, i_tr, i_bl, i_br])
    w4 = jnp.stack([w_tl, w_tr, w_bl, w_br])

    outs = []
    for b in range(B):
        meta = idx4[:, b].reshape(4, _NW, chunks, _K).transpose(1, 2, 0, 3)
        meta = jnp.pad(meta, ((0, 0), (0, 2), (0, 0), (0, 0)))
        wts = w4[:, b].reshape(4, _NW, chunks, _K).transpose(1, 2, 0, 3)
        wts = jnp.pad(wts, ((0, 0), (0, 2), (0, 0), (0, 0)))
        xt = jnp.transpose(x[b], (1, 2, 0)).reshape(N, C)
        xt = jnp.pad(xt, ((0, 0), (0, _CP - C)))
        out_t = _interp_sc(xt, meta, wts, N, C)
        outs.append(
            out_t.reshape(H, W, C).transpose(2, 0, 1) + residual[b])
    return jnp.stack(outs)


# speedup vs baseline: 17.3520x; 1.3880x over previous
import functools

import jax
import jax.numpy as jnp
from jax import lax
from jax.experimental import pallas as pl
from jax.experimental.pallas import tpu as pltpu
from jax.experimental.pallas import tpu_sc as plsc

_NC = 2
_NS = 16
_NW = _NC * _NS
_K = 64
_L = 16
_CP = 128


def _interp_sc(xt, meta, wts, N, C):
    per_w = N // _NW
    chunks = per_w // _K
    mesh = plsc.VectorSubcoreMesh(core_axis_name="c", subcore_axis_name="s")

    taps_t = pltpu.VMEM((_K, _CP), jnp.float32)

    @functools.partial(
        pl.kernel,
        out_type=jax.ShapeDtypeStruct((N, C), jnp.float32),
        mesh=mesh,
        compiler_params=pltpu.CompilerParams(use_tc_tiling_on_sc=True),
        scratch_types=[
            pltpu.VMEM((4, _K), jnp.int32),
            pltpu.VMEM((4, _K), jnp.int32),
            pltpu.VMEM((4, _K), jnp.float32),
            pltpu.VMEM((4, _K), jnp.float32),
            taps_t, taps_t, taps_t, taps_t,
            taps_t, taps_t, taps_t, taps_t,
            pltpu.VMEM((_K, C), jnp.float32),
            pltpu.VMEM((_K, C), jnp.float32),
            pltpu.SemaphoreType.DMA,
            pltpu.SemaphoreType.DMA,
            pltpu.SemaphoreType.DMA,
            pltpu.SemaphoreType.DMA,
            pltpu.SemaphoreType.DMA,
            pltpu.SemaphoreType.DMA,
            pltpu.SemaphoreType.DMA,
            pltpu.SemaphoreType.DMA,
        ],
    )
    def k(xt_hbm, meta_hbm, w_hbm, out_hbm, m0, m1, w0, w1,
          ta0, ta1, ta2, ta3, tb0, tb1, tb2, tb3, ov0, ov1,
          sem_ga, sem_gb, sem_ma, sem_mb, sem_wa, sem_wb, sem_oa, sem_ob):
        wid = lax.axis_index("s") * _NC + lax.axis_index("c")
        base0 = wid * per_w
        metas = (m0, m1)
        wvs = (w0, w1)
        taps = ((ta0, ta1, ta2, ta3), (tb0, tb1, tb2, tb3))
        outs = (ov0, ov1)
        gsems = (sem_ga, sem_gb)
        msems = (sem_ma, sem_mb)
        wsems = (sem_wa, sem_wb)
        osems = (sem_oa, sem_ob)

        def wait_gathers(p):
            for t in range(4):
                pltpu.make_async_copy(
                    xt_hbm.at[metas[p].at[t]], taps[p][t], gsems[p]).wait()

        def issue_gathers(p):
            for t in range(4):
                pltpu.async_copy(
                    xt_hbm.at[metas[p].at[t]], taps[p][t], gsems[p])

        def issue_meta(g, p):
            pltpu.async_copy(meta_hbm.at[wid, g], metas[p], msems[p])

        def wait_meta(p):
            pltpu.make_async_copy(
                meta_hbm.at[wid, 0], metas[p], msems[p]).wait()

        def issue_w(g, p):
            pltpu.async_copy(w_hbm.at[wid, g], wvs[p], wsems[p])

        def wait_w(p):
            pltpu.make_async_copy(
                w_hbm.at[wid, 0], wvs[p], wsems[p]).wait()

        def wait_out(p):
            pltpu.make_async_copy(
                outs[p], out_hbm.at[pl.ds(0, _K)], osems[p]).wait()

        def do_chunk(g, i, p, guard_out):
            q = 1 - p
            wait_gathers(p)
            wait_meta(q)
            issue_gathers(q)
            issue_meta(g + 2, p)
            wait_w(p)
            if guard_out is None:
                wait_out(p)
            else:
                @pl.when(guard_out)
                def _():
                    wait_out(p)
            tp = taps[p]
            w_v = wvs[p]
            out_v = outs[p]

            def grp_body(g2, carry):
                bp = g2 * _L
                wv = [w_v[t, pl.ds(bp, _L)] for t in range(4)]
                for ii in range(_L):
                    pix = bp + ii
                    ws = [wv[t][ii] for t in range(4)]
                    for j in range(C // _L):
                        s = pl.ds(j * _L, _L)
                        acc = ws[0] * tp[0][pix, s] + ws[1] * tp[1][pix, s]
                        acc = acc + ws[2] * tp[2][pix, s] + ws[3] * tp[3][pix, s]
                        out_v[pix, s] = acc
                return carry

            lax.fori_loop(0, _K // _L, grp_body, 0)
            issue_w(g + 2, p)
            pltpu.async_copy(
                out_v, out_hbm.at[pl.ds(base0 + g * _K, _K)], osems[p])

        issue_meta(0, 0)
        issue_meta(1, 1)
        issue_w(0, 0)
        issue_w(1, 1)
        wait_meta(0)
        issue_gathers(0)

        def pair_body(i, carry):
            do_chunk(2 * i, i, 0, i > 0)
            do_chunk(2 * i + 1, i, 1, i > 0)
            return carry

        lax.fori_loop(0, chunks // 2, pair_body, 0)
        wait_gathers(0)
        wait_meta(1)
        wait_w(0)
        wait_w(1)
        wait_out(0)
        wait_out(1)

    return k(xt, meta, wts)


def _epilogue_tc(out_t, residual):
    B, C, H, W = residual.shape

    HB = 8

    def body(t_ref, r_ref, o_ref):
        t = t_ref[...]
        o_ref[...] = (
            jnp.transpose(t, (1, 0)).reshape(1, C, HB, W) + r_ref[...])

    return pl.pallas_call(
        body,
        grid=(B, H // HB),
        in_specs=[
            pl.BlockSpec((HB * W, C), lambda b, h: (b * (H // HB) + h, 0)),
            pl.BlockSpec((1, C, HB, W), lambda b, h: (b, 0, h, 0)),
        ],
        out_specs=pl.BlockSpec((1, C, HB, W), lambda b, h: (b, 0, h, 0)),
        out_shape=jax.ShapeDtypeStruct((B, C, H, W), jnp.float32),
    )(out_t, residual)


def kernel(input, flow, residual):
    x = input
    B, C, H, W = x.shape
    N = B * H * W
    per_w = N // _NW
    chunks = per_w // _K

    gy = jnp.linspace(-1.0 + 1.0 / H, 1.0 - 1.0 / H, H, dtype=x.dtype)
    gx = jnp.linspace(-1.0 + 1.0 / W, 1.0 - 1.0 / W, W, dtype=x.dtype)
    grid_x = gx[None, None, :] + flow[:, 0]
    grid_y = gy[None, :, None] + flow[:, 1]
    grid_x = jnp.remainder(grid_x + 1.0, 2.0) - 1.0
    real_x = (grid_x + 1.0) * (W * 0.5) - 0.5
    real_y = (grid_y + 1.0) * (H * 0.5) - 0.5
    x0f = jnp.floor(real_x)
    y0f = jnp.floor(real_y)
    dx = real_x - x0f
    dy = real_y - y0f
    ix0 = x0f.astype(jnp.int32)
    iy0 = y0f.astype(jnp.int32)
    boff = (jnp.arange(B, dtype=jnp.int32) * (H * W))[:, None, None]

    def idx_w(iy, ix, w):
        valid = (ix >= 0) & (ix < W) & (iy >= 0) & (iy < H)
        iyc = jnp.clip(iy, 0, H - 1)
        ixc = jnp.clip(ix, 0, W - 1)
        idx = iyc * W + ixc + boff
        return idx.reshape(N), (w * valid).reshape(N)

    i_tl, w_tl = idx_w(iy0, ix0, (1.0 - dx) * (1.0 - dy))
    i_tr, w_tr = idx_w(iy0, ix0 + 1, dx * (1.0 - dy))
    i_bl, w_bl = idx_w(iy0 + 1, ix0, (1.0 - dx) * dy)
    i_br, w_br = idx_w(iy0 + 1, ix0 + 1, dx * dy)
    idx4 = jnp.stack([i_tl, i_tr, i_bl, i_br])
    w4 = jnp.stack([w_tl, w_tr, w_bl, w_br])
    meta = idx4.reshape(4, _NW, chunks, _K).transpose(1, 2, 0, 3)
    meta = jnp.pad(meta, ((0, 0), (0, 2), (0, 0), (0, 0)))
    wts = w4.reshape(4, _NW, chunks, _K).transpose(1, 2, 0, 3)
    wts = jnp.pad(wts, ((0, 0), (0, 2), (0, 0), (0, 0)))

    xt = jnp.transpose(x, (0, 2, 3, 1)).reshape(N, C)
    xt = jnp.pad(xt, ((0, 0), (0, _CP - C)))
    out_t = _interp_sc(xt, meta, wts, N, C)
    return _epilogue_tc(out_t, residual)


# speedup vs baseline: 17.6468x; 1.0170x over previous
import functools

import jax
import jax.numpy as jnp
from jax import lax
from jax.experimental import pallas as pl
from jax.experimental.pallas import tpu as pltpu
from jax.experimental.pallas import tpu_sc as plsc

_NC = 2
_NS = 16
_NW = _NC * _NS
_K = 64
_L = 16
_CP = 128


def _interp_sc(xt, meta, wts, N, C):
    per_w = N // _NW
    chunks = per_w // _K
    mesh = plsc.VectorSubcoreMesh(core_axis_name="c", subcore_axis_name="s")

    taps_t = pltpu.VMEM((_K, _CP), jnp.float32)

    @functools.partial(
        pl.kernel,
        out_type=jax.ShapeDtypeStruct((N, C), jnp.float32),
        mesh=mesh,
        compiler_params=pltpu.CompilerParams(use_tc_tiling_on_sc=True),
        scratch_types=[
            pltpu.VMEM((4, _K), jnp.int32),
            pltpu.VMEM((4, _K), jnp.int32),
            pltpu.VMEM((4, _K), jnp.float32),
            pltpu.VMEM((4, _K), jnp.float32),
            taps_t, taps_t, taps_t, taps_t,
            taps_t, taps_t, taps_t, taps_t,
            pltpu.VMEM((_K, C), jnp.float32),
            pltpu.VMEM((_K, C), jnp.float32),
            pltpu.SemaphoreType.DMA,
            pltpu.SemaphoreType.DMA,
            pltpu.SemaphoreType.DMA,
            pltpu.SemaphoreType.DMA,
            pltpu.SemaphoreType.DMA,
            pltpu.SemaphoreType.DMA,
            pltpu.SemaphoreType.DMA,
            pltpu.SemaphoreType.DMA,
        ],
    )
    def k(xt_hbm, meta_hbm, w_hbm, out_hbm, m0, m1, w0, w1,
          ta0, ta1, ta2, ta3, tb0, tb1, tb2, tb3, ov0, ov1,
          sem_ga, sem_gb, sem_ma, sem_mb, sem_wa, sem_wb, sem_oa, sem_ob):
        wid = lax.axis_index("s") * _NC + lax.axis_index("c")
        base0 = wid * per_w
        metas = (m0, m1)
        wvs = (w0, w1)
        taps = ((ta0, ta1, ta2, ta3), (tb0, tb1, tb2, tb3))
        outs = (ov0, ov1)
        gsems = (sem_ga, sem_gb)
        msems = (sem_ma, sem_mb)
        wsems = (sem_wa, sem_wb)
        osems = (sem_oa, sem_ob)

        def wait_gathers(p):
            for t in range(4):
                pltpu.make_async_copy(
                    xt_hbm.at[metas[p].at[t]], taps[p][t], gsems[p]).wait()

        def issue_gathers(p):
            for t in range(4):
                pltpu.async_copy(
                    xt_hbm.at[metas[p].at[t]], taps[p][t], gsems[p])

        def issue_meta(g, p):
            pltpu.async_copy(meta_hbm.at[wid, g], metas[p], msems[p])

        def wait_meta(p):
            pltpu.make_async_copy(
                meta_hbm.at[wid, 0], metas[p], msems[p]).wait()

        def issue_w(g, p):
            pltpu.async_copy(w_hbm.at[wid, g], wvs[p], wsems[p])

        def wait_w(p):
            pltpu.make_async_copy(
                w_hbm.at[wid, 0], wvs[p], wsems[p]).wait()

        def wait_out(p):
            pltpu.make_async_copy(
                outs[p], out_hbm.at[pl.ds(0, _K)], osems[p]).wait()

        def do_chunk(g, i, p, guard_out):
            q = 1 - p
            wait_gathers(p)
            wait_meta(q)
            issue_gathers(q)
            issue_meta(g + 2, p)
            wait_w(p)
            if guard_out is None:
                wait_out(p)
            else:
                @pl.when(guard_out)
                def _():
                    wait_out(p)
            tp = taps[p]
            w_v = wvs[p]
            out_v = outs[p]

            def grp_body(g2, carry):
                bp = g2 * _L
                wv = [w_v[t, pl.ds(bp, _L)] for t in range(4)]
                for ii in range(_L):
                    pix = bp + ii
                    ws = [wv[t][ii] for t in range(4)]
                    for j in range(C // _L):
                        s = pl.ds(j * _L, _L)
                        acc = ws[0] * tp[0][pix, s] + ws[1] * tp[1][pix, s]
                        acc = acc + ws[2] * tp[2][pix, s] + ws[3] * tp[3][pix, s]
                        out_v[pix, s] = acc
                return carry

            lax.fori_loop(0, _K // _L, grp_body, 0)
            issue_w(g + 2, p)
            pltpu.async_copy(
                out_v, out_hbm.at[pl.ds(base0 + g * _K, _K)], osems[p])

        issue_meta(0, 0)
        issue_meta(1, 1)
        issue_w(0, 0)
        issue_w(1, 1)
        wait_meta(0)
        issue_gathers(0)

        def pair_body(i, carry):
            do_chunk(2 * i, i, 0, i > 0)
            do_chunk(2 * i + 1, i, 1, i > 0)
            return carry

        lax.fori_loop(0, chunks // 2, pair_body, 0)
        wait_gathers(0)
        wait_meta(1)
        wait_w(0)
        wait_w(1)
        wait_out(0)
        wait_out(1)

    return k(xt, meta, wts)


def _epilogue_tc(out_t, residual):
    B, C, H, W = residual.shape

    HB = 8

    def body(t_ref, r_ref, o_ref):
        t = t_ref[...]
        o_ref[...] = (
            jnp.transpose(t, (1, 0)).reshape(1, C, HB, W) + r_ref[...])

    return pl.pallas_call(
        body,
        grid=(B, H // HB),
        in_specs=[
            pl.BlockSpec((HB * W, C), lambda b, h: (b * (H // HB) + h, 0)),
            pl.BlockSpec((1, C, HB, W), lambda b, h: (b, 0, h, 0)),
        ],
        out_specs=pl.BlockSpec((1, C, HB, W), lambda b, h: (b, 0, h, 0)),
        out_shape=jax.ShapeDtypeStruct((B, C, H, W), jnp.float32),
    )(out_t, residual)


def _prologue_tc(x):
    B, C, H, W = x.shape
    HB = 8

    def body(x_ref, o_ref):
        t = x_ref[...].reshape(C, HB * W)
        tt = jnp.transpose(t, (1, 0))
        o_ref[...] = jnp.concatenate(
            [tt, jnp.zeros((HB * W, _CP - C), jnp.float32)], axis=1)

    return pl.pallas_call(
        body,
        grid=(B, H // HB),
        in_specs=[pl.BlockSpec((1, C, HB, W), lambda b, h: (b, 0, h, 0))],
        out_specs=pl.BlockSpec(
            (HB * W, _CP), lambda b, h: (b * (H // HB) + h, 0)),
        out_shape=jax.ShapeDtypeStruct((B * H * W, _CP), jnp.float32),
    )(x)


def kernel(input, flow, residual):
    x = input
    B, C, H, W = x.shape
    N = B * H * W
    per_w = N // _NW
    chunks = per_w // _K

    gy = jnp.linspace(-1.0 + 1.0 / H, 1.0 - 1.0 / H, H, dtype=x.dtype)
    gx = jnp.linspace(-1.0 + 1.0 / W, 1.0 - 1.0 / W, W, dtype=x.dtype)
    grid_x = gx[None, None, :] + flow[:, 0]
    grid_y = gy[None, :, None] + flow[:, 1]
    grid_x = jnp.remainder(grid_x + 1.0, 2.0) - 1.0
    real_x = (grid_x + 1.0) * (W * 0.5) - 0.5
    real_y = (grid_y + 1.0) * (H * 0.5) - 0.5
    x0f = jnp.floor(real_x)
    y0f = jnp.floor(real_y)
    dx = real_x - x0f
    dy = real_y - y0f
    ix0 = x0f.astype(jnp.int32)
    iy0 = y0f.astype(jnp.int32)
    boff = (jnp.arange(B, dtype=jnp.int32) * (H * W))[:, None, None]

    def idx_w(iy, ix, w):
        valid = (ix >= 0) & (ix < W) & (iy >= 0) & (iy < H)
        iyc = jnp.clip(iy, 0, H - 1)
        ixc = jnp.clip(ix, 0, W - 1)
        idx = iyc * W + ixc + boff
        return idx.reshape(N), (w * valid).reshape(N)

    i_tl, w_tl = idx_w(iy0, ix0, (1.0 - dx) * (1.0 - dy))
    i_tr, w_tr = idx_w(iy0, ix0 + 1, dx * (1.0 - dy))
    i_bl, w_bl = idx_w(iy0 + 1, ix0, (1.0 - dx) * dy)
    i_br, w_br = idx_w(iy0 + 1, ix0 + 1, dx * dy)
    idx4 = jnp.stack([i_tl, i_tr, i_bl, i_br])
    w4 = jnp.stack([w_tl, w_tr, w_bl, w_br])
    meta = idx4.reshape(4, _NW, chunks, _K).transpose(1, 2, 0, 3)
    meta = jnp.pad(meta, ((0, 0), (0, 2), (0, 0), (0, 0)))
    wts = w4.reshape(4, _NW, chunks, _K).transpose(1, 2, 0, 3)
    wts = jnp.pad(wts, ((0, 0), (0, 2), (0, 0), (0, 0)))

    xt = _prologue_tc(x)
    out_t = _interp_sc(xt, meta, wts, N, C)
    return _epilogue_tc(out_t, residual)


# speedup vs baseline: 17.9202x; 1.0155x over previous
import functools

import jax
import jax.numpy as jnp
from jax import lax
from jax.experimental import pallas as pl
from jax.experimental.pallas import tpu as pltpu
from jax.experimental.pallas import tpu_sc as plsc

_NC = 2
_NS = 16
_NW = _NC * _NS
_K = 64
_L = 16
_CP = 128


def _interp_sc(xt, meta, wts, N, C):
    per_w = N // _NW
    chunks = per_w // _K
    mesh = plsc.VectorSubcoreMesh(core_axis_name="c", subcore_axis_name="s")

    taps_t = pltpu.VMEM((2 * _K, _CP), jnp.float32)

    @functools.partial(
        pl.kernel,
        out_type=jax.ShapeDtypeStruct((N, C), jnp.float32),
        mesh=mesh,
        compiler_params=pltpu.CompilerParams(use_tc_tiling_on_sc=True),
        scratch_types=[
            pltpu.VMEM((2, 2 * _K), jnp.int32),
            pltpu.VMEM((2, 2 * _K), jnp.int32),
            pltpu.VMEM((4, _K), jnp.float32),
            pltpu.VMEM((4, _K), jnp.float32),
            taps_t, taps_t,
            taps_t, taps_t,
            pltpu.VMEM((_K, C), jnp.float32),
            pltpu.VMEM((_K, C), jnp.float32),
            pltpu.SemaphoreType.DMA,
            pltpu.SemaphoreType.DMA,
            pltpu.SemaphoreType.DMA,
            pltpu.SemaphoreType.DMA,
            pltpu.SemaphoreType.DMA,
            pltpu.SemaphoreType.DMA,
            pltpu.SemaphoreType.DMA,
            pltpu.SemaphoreType.DMA,
        ],
    )
    def k(xt_hbm, meta_hbm, w_hbm, out_hbm, m0, m1, w0, w1,
          ta0, ta1, tb0, tb1, ov0, ov1,
          sem_ga, sem_gb, sem_ma, sem_mb, sem_wa, sem_wb, sem_oa, sem_ob):
        wid = lax.axis_index("s") * _NC + lax.axis_index("c")
        base0 = wid * per_w
        metas = (m0, m1)
        wvs = (w0, w1)
        taps = ((ta0, ta1), (tb0, tb1))
        outs = (ov0, ov1)
        gsems = (sem_ga, sem_gb)
        msems = (sem_ma, sem_mb)
        wsems = (sem_wa, sem_wb)
        osems = (sem_oa, sem_ob)

        def wait_gathers(p):
            for t in range(2):
                pltpu.make_async_copy(
                    xt_hbm.at[metas[p].at[t]], taps[p][t], gsems[p]).wait()

        def issue_gathers(p):
            for t in range(2):
                pltpu.async_copy(
                    xt_hbm.at[metas[p].at[t]], taps[p][t], gsems[p])

        def issue_meta(g, p):
            pltpu.async_copy(meta_hbm.at[wid, g], metas[p], msems[p])

        def wait_meta(p):
            pltpu.make_async_copy(
                meta_hbm.at[wid, 0], metas[p], msems[p]).wait()

        def issue_w(g, p):
            pltpu.async_copy(w_hbm.at[wid, g], wvs[p], wsems[p])

        def wait_w(p):
            pltpu.make_async_copy(
                w_hbm.at[wid, 0], wvs[p], wsems[p]).wait()

        def wait_out(p):
            pltpu.make_async_copy(
                outs[p], out_hbm.at[pl.ds(0, _K)], osems[p]).wait()

        def do_chunk(g, i, p, guard_out):
            q = 1 - p
            wait_gathers(p)
            wait_meta(q)
            issue_gathers(q)
            issue_meta(g + 2, p)
            wait_w(p)
            if guard_out is None:
                wait_out(p)
            else:
                @pl.when(guard_out)
                def _():
                    wait_out(p)
            top, bot = taps[p]
            w_v = wvs[p]
            out_v = outs[p]

            def grp_body(g2, carry):
                bp = g2 * _L
                wv = [w_v[t, pl.ds(bp, _L)] for t in range(4)]
                for ii in range(_L):
                    pix = bp + ii
                    ws = [wv[t][ii] for t in range(4)]
                    for j in range(C // _L):
                        s = pl.ds(j * _L, _L)
                        acc = ws[0] * top[pix, s] + ws[1] * top[_K + pix, s]
                        acc = acc + ws[2] * bot[pix, s] + ws[3] * bot[_K + pix, s]
                        out_v[pix, s] = acc
                return carry

            lax.fori_loop(0, _K // _L, grp_body, 0)
            issue_w(g + 2, p)
            pltpu.async_copy(
                out_v, out_hbm.at[pl.ds(base0 + g * _K, _K)], osems[p])

        issue_meta(0, 0)
        issue_meta(1, 1)
        issue_w(0, 0)
        issue_w(1, 1)
        wait_meta(0)
        issue_gathers(0)

        def pair_body(i, carry):
            do_chunk(2 * i, i, 0, i > 0)
            do_chunk(2 * i + 1, i, 1, i > 0)
            return carry

        lax.fori_loop(0, chunks // 2, pair_body, 0)
        wait_gathers(0)
        wait_meta(1)
        wait_w(0)
        wait_w(1)
        wait_out(0)
        wait_out(1)

    return k(xt, meta, wts)


def _epilogue_tc(out_t, residual):
    B, C, H, W = residual.shape

    HB = 8

    def body(t_ref, r_ref, o_ref):
        t = t_ref[...]
        o_ref[...] = (
            jnp.transpose(t, (1, 0)).reshape(1, C, HB, W) + r_ref[...])

    return pl.pallas_call(
        body,
        grid=(B, H // HB),
        in_specs=[
            pl.BlockSpec((HB * W, C), lambda b, h: (b * (H // HB) + h, 0)),
            pl.BlockSpec((1, C, HB, W), lambda b, h: (b, 0, h, 0)),
        ],
        out_specs=pl.BlockSpec((1, C, HB, W), lambda b, h: (b, 0, h, 0)),
        out_shape=jax.ShapeDtypeStruct((B, C, H, W), jnp.float32),
    )(out_t, residual)


def _prologue_tc(x):
    B, C, H, W = x.shape
    HB = 8

    def body(x_ref, o_ref):
        t = x_ref[...].reshape(C, HB * W)
        tt = jnp.transpose(t, (1, 0))
        o_ref[...] = jnp.concatenate(
            [tt, jnp.zeros((HB * W, _CP - C), jnp.float32)], axis=1)

    return pl.pallas_call(
        body,
        grid=(B, H // HB),
        in_specs=[pl.BlockSpec((1, C, HB, W), lambda b, h: (b, 0, h, 0))],
        out_specs=pl.BlockSpec(
            (HB * W, _CP), lambda b, h: (b * (H // HB) + h, 0)),
        out_shape=jax.ShapeDtypeStruct((B * H * W, _CP), jnp.float32),
    )(x)


def kernel(input, flow, residual):
    x = input
    B, C, H, W = x.shape
    N = B * H * W
    per_w = N // _NW
    chunks = per_w // _K

    gy = jnp.linspace(-1.0 + 1.0 / H, 1.0 - 1.0 / H, H, dtype=x.dtype)
    gx = jnp.linspace(-1.0 + 1.0 / W, 1.0 - 1.0 / W, W, dtype=x.dtype)
    grid_x = gx[None, None, :] + flow[:, 0]
    grid_y = gy[None, :, None] + flow[:, 1]
    grid_x = jnp.remainder(grid_x + 1.0, 2.0) - 1.0
    real_x = (grid_x + 1.0) * (W * 0.5) - 0.5
    real_y = (grid_y + 1.0) * (H * 0.5) - 0.5
    x0f = jnp.floor(real_x)
    y0f = jnp.floor(real_y)
    dx = real_x - x0f
    dy = real_y - y0f
    ix0 = x0f.astype(jnp.int32)
    iy0 = y0f.astype(jnp.int32)
    boff = (jnp.arange(B, dtype=jnp.int32) * (H * W))[:, None, None]

    def idx_w(iy, ix, w):
        valid = (ix >= 0) & (ix < W) & (iy >= 0) & (iy < H)
        iyc = jnp.clip(iy, 0, H - 1)
        ixc = jnp.clip(ix, 0, W - 1)
        idx = iyc * W + ixc + boff
        return idx.reshape(N), (w * valid).reshape(N)

    i_tl, w_tl = idx_w(iy0, ix0, (1.0 - dx) * (1.0 - dy))
    i_tr, w_tr = idx_w(iy0, ix0 + 1, dx * (1.0 - dy))
    i_bl, w_bl = idx_w(iy0 + 1, ix0, (1.0 - dx) * dy)
    i_br, w_br = idx_w(iy0 + 1, ix0 + 1, dx * dy)
    idx4 = jnp.stack([i_tl, i_tr, i_bl, i_br])
    w4 = jnp.stack([w_tl, w_tr, w_bl, w_br])
    meta = idx4.reshape(4, _NW, chunks, _K).transpose(1, 2, 0, 3)
    meta = meta.reshape(_NW, chunks, 2, 2 * _K)
    meta = jnp.pad(meta, ((0, 0), (0, 2), (0, 0), (0, 0)))
    wts = w4.reshape(4, _NW, chunks, _K).transpose(1, 2, 0, 3)
    wts = jnp.pad(wts, ((0, 0), (0, 2), (0, 0), (0, 0)))

    xt = _prologue_tc(x)
    out_t = _interp_sc(xt, meta, wts, N, C)
    return _epilogue_tc(out_t, residual)
